# Initial kernel scaffold; baseline (speedup 1.0000x reference)
import jax
import jax.numpy as jnp
from jax.experimental import pallas as pl


def kernel(batch_token, edge_p_node, edge_c_node, edge_p_indicate, edge_c_indicate, p_mask, c_mask, start_token, end_token, Wv1, bv1, Wv2, bv2, We1, be1, We2, be2, Wp1, bp1, Wp2, bp2, Wc1, bc1, Wc2, bc2, Wa1, ba1, Wa2, ba2):
    raise NotImplementedError("write your pallas kernel here")


# speedup vs baseline: 1.8841x; 1.8841x over previous
import functools

import jax
import jax.numpy as jnp
from jax import lax
from jax.experimental import pallas as pl
from jax.experimental.pallas import tpu as pltpu
from jax.experimental.pallas import tpu_sc as plsc

N = 10000
E = 320000
D = 128
H1 = 256
H2 = 128
NUM_HOPS = 3

N_PAD = 10240
NS = 16
PER_TILE = E // NS
CH = 80
NCHUNK = PER_TILE // CH
ROWS_PER_TILE = N_PAD // NS

BLK_N = 2048
BLK_E = 1280

_f32 = jnp.float32


def _node_mlp_body(x_ref, w1_ref, b1_ref, w2_ref, b2_ref, o_ref):
    h = jnp.maximum(
        jnp.dot(x_ref[...], w1_ref[...], preferred_element_type=_f32)
        + b1_ref[...], 0.0)
    o_ref[...] = jnp.maximum(
        jnp.dot(h, w2_ref[...], preferred_element_type=_f32)
        + b2_ref[...], 0.0)


def _node_mlp(x, w1, b1, w2, b2):
    n = x.shape[0]
    grid = n // BLK_N
    return pl.pallas_call(
        _node_mlp_body,
        grid=(grid,),
        in_specs=[
            pl.BlockSpec((BLK_N, D), lambda i: (i, 0)),
            pl.BlockSpec((D, H1), lambda i: (0, 0)),
            pl.BlockSpec((1, H1), lambda i: (0, 0)),
            pl.BlockSpec((H1, H2), lambda i: (0, 0)),
            pl.BlockSpec((1, H2), lambda i: (0, 0)),
        ],
        out_specs=pl.BlockSpec((BLK_N, H2), lambda i: (i, 0)),
        out_shape=jax.ShapeDtypeStruct((n, H2), _f32),
    )(x, w1, b1, w2, b2)


def _edge_pre_body(tp_ref, tc_ref, we1_ref, be1_ref, we2_ref, be2_ref,
                   wpe_ref, bp1_ref, wce_ref, bc1_ref, prep_ref, prec_ref):
    def one(t_ref, we_ref, b_ref):
        h = jnp.maximum(t_ref[...] * we1_ref[...] + be1_ref[...], 0.0)
        e = jnp.maximum(
            jnp.dot(h, we2_ref[...], preferred_element_type=_f32)
            + be2_ref[...], 0.0)
        return jnp.dot(e, we_ref[...], preferred_element_type=_f32) + b_ref[...]
    prep_ref[...] = one(tp_ref, wpe_ref, bp1_ref)
    prec_ref[...] = one(tc_ref, wce_ref, bc1_ref)


def _edge_pre(tp, tc, we1, be1, we2, be2, wpe, bp1, wce, bc1):
    grid = E // BLK_E
    return pl.pallas_call(
        _edge_pre_body,
        grid=(grid,),
        in_specs=[
            pl.BlockSpec((BLK_E, 1), lambda i: (i, 0)),
            pl.BlockSpec((BLK_E, 1), lambda i: (i, 0)),
            pl.BlockSpec((1, H1), lambda i: (0, 0)),
            pl.BlockSpec((1, H1), lambda i: (0, 0)),
            pl.BlockSpec((H1, H2), lambda i: (0, 0)),
            pl.BlockSpec((1, H2), lambda i: (0, 0)),
            pl.BlockSpec((H2, H1), lambda i: (0, 0)),
            pl.BlockSpec((1, H1), lambda i: (0, 0)),
            pl.BlockSpec((H2, H1), lambda i: (0, 0)),
            pl.BlockSpec((1, H1), lambda i: (0, 0)),
        ],
        out_specs=[
            pl.BlockSpec((BLK_E, H1), lambda i: (i, 0)),
            pl.BlockSpec((BLK_E, H1), lambda i: (i, 0)),
        ],
        out_shape=[
            jax.ShapeDtypeStruct((E, H1), _f32),
            jax.ShapeDtypeStruct((E, H1), _f32),
        ],
    )(tp, tc, we1, be1, we2, be2, wpe, bp1, wce, bc1)


def _edge_mlp_body(hp_ref, hc_ref, prep_ref, prec_ref,
                   wp1c_ref, wp1p_ref, wp2_ref, bp2_ref,
                   wc1p_ref, wc1c_ref, wc2_ref, bc2_ref,
                   sp_ref, sc_ref):
    hp = hp_ref[...]
    hc = hc_ref[...]
    h1p = jnp.maximum(
        jnp.dot(hc, wp1c_ref[...], preferred_element_type=_f32)
        + jnp.dot(hp, wp1p_ref[...], preferred_element_type=_f32)
        + prep_ref[...], 0.0)
    sp_ref[...] = jnp.maximum(
        jnp.dot(h1p, wp2_ref[...], preferred_element_type=_f32)
        + bp2_ref[...], 0.0)
    h1c = jnp.maximum(
        jnp.dot(hp, wc1p_ref[...], preferred_element_type=_f32)
        + jnp.dot(hc, wc1c_ref[...], preferred_element_type=_f32)
        + prec_ref[...], 0.0)
    sc_ref[...] = jnp.maximum(
        jnp.dot(h1c, wc2_ref[...], preferred_element_type=_f32)
        + bc2_ref[...], 0.0)


def _edge_mlp(hp, hc, prep, prec, wp1c, wp1p, wp2, bp2, wc1p, wc1c, wc2, bc2):
    grid = E // BLK_E
    return pl.pallas_call(
        _edge_mlp_body,
        grid=(grid,),
        in_specs=[
            pl.BlockSpec((BLK_E, H2), lambda i: (i, 0)),
            pl.BlockSpec((BLK_E, H2), lambda i: (i, 0)),
            pl.BlockSpec((BLK_E, H1), lambda i: (i, 0)),
            pl.BlockSpec((BLK_E, H1), lambda i: (i, 0)),
            pl.BlockSpec((H2, H1), lambda i: (0, 0)),
            pl.BlockSpec((H2, H1), lambda i: (0, 0)),
            pl.BlockSpec((H1, H2), lambda i: (0, 0)),
            pl.BlockSpec((1, H2), lambda i: (0, 0)),
            pl.BlockSpec((H2, H1), lambda i: (0, 0)),
            pl.BlockSpec((H2, H1), lambda i: (0, 0)),
            pl.BlockSpec((H1, H2), lambda i: (0, 0)),
            pl.BlockSpec((1, H2), lambda i: (0, 0)),
        ],
        out_specs=[
            pl.BlockSpec((BLK_E, H2), lambda i: (i, 0)),
            pl.BlockSpec((BLK_E, H2), lambda i: (i, 0)),
        ],
        out_shape=[
            jax.ShapeDtypeStruct((E, H2), _f32),
            jax.ShapeDtypeStruct((E, H2), _f32),
        ],
    )(hp, hc, prep, prec, wp1c, wp1p, wp2, bp2, wc1p, wc1c, wc2, bc2)


def _aggr_body(hid_ref, sump_ref, sumc_ref, cntp_ref, cntc_ref,
               pm_ref, cm_ref, st_ref, et_ref,
               wa1h_ref, wa1p_ref, wa1c_ref, ba1_ref, wa2_ref, ba2_ref,
               o_ref):
    hid = hid_ref[...]
    s_p = sump_ref[...] / jnp.maximum(cntp_ref[:, :1], 1.0) \
        + pm_ref[...] * st_ref[...]
    s_c = sumc_ref[...] / jnp.maximum(cntc_ref[:, :1], 1.0) \
        + cm_ref[...] * et_ref[...]
    h1 = jnp.maximum(
        jnp.dot(hid, wa1h_ref[...], preferred_element_type=_f32)
        + jnp.dot(s_p, wa1p_ref[...], preferred_element_type=_f32)
        + jnp.dot(s_c, wa1c_ref[...], preferred_element_type=_f32)
        + ba1_ref[...], 0.0)
    o_ref[...] = hid + jnp.maximum(
        jnp.dot(h1, wa2_ref[...], preferred_element_type=_f32)
        + ba2_ref[...], 0.0)


def _aggr(hid, sump, sumc, cntp, cntc, pm, cm, st, et,
          wa1h, wa1p, wa1c, ba1, wa2, ba2):
    grid = N_PAD // BLK_N
    return pl.pallas_call(
        _aggr_body,
        grid=(grid,),
        in_specs=[
            pl.BlockSpec((BLK_N, H2), lambda i: (i, 0)),
            pl.BlockSpec((BLK_N, H2), lambda i: (i, 0)),
            pl.BlockSpec((BLK_N, H2), lambda i: (i, 0)),
            pl.BlockSpec((BLK_N, H2), lambda i: (i, 0)),
            pl.BlockSpec((BLK_N, H2), lambda i: (i, 0)),
            pl.BlockSpec((BLK_N, 1), lambda i: (i, 0)),
            pl.BlockSpec((BLK_N, 1), lambda i: (i, 0)),
            pl.BlockSpec((1, D), lambda i: (0, 0)),
            pl.BlockSpec((1, D), lambda i: (0, 0)),
            pl.BlockSpec((H2, H1), lambda i: (0, 0)),
            pl.BlockSpec((H2, H1), lambda i: (0, 0)),
            pl.BlockSpec((H2, H1), lambda i: (0, 0)),
            pl.BlockSpec((1, H1), lambda i: (0, 0)),
            pl.BlockSpec((H1, H2), lambda i: (0, 0)),
            pl.BlockSpec((1, H2), lambda i: (0, 0)),
        ],
        out_specs=pl.BlockSpec((BLK_N, H2), lambda i: (i, 0)),
        out_shape=jax.ShapeDtypeStruct((N_PAD, H2), _f32),
    )(hid, sump, sumc, cntp, cntc, pm, cm, st, et,
      wa1h, wa1p, wa1c, ba1, wa2, ba2)


def _sc_mesh():
    return plsc.VectorSubcoreMesh(core_axis_name="c", subcore_axis_name="s",
                                  num_cores=2, num_subcores=NS)


def _gather_sc(hid, idx_p, idx_c):
    @functools.partial(
        pl.kernel,
        out_type=[
            jax.ShapeDtypeStruct((E, H2), _f32),
            jax.ShapeDtypeStruct((E, H2), _f32),
        ],
        mesh=_sc_mesh(),
        scratch_types=[
            pltpu.VMEM((CH,), jnp.int32),
            pltpu.VMEM((CH, H2), _f32),
            pltpu.SemaphoreType.DMA,
        ],
    )
    def k(hid_hbm, idxp_hbm, idxc_hbm, outp_hbm, outc_hbm, idx_v, rows_v, sem):
        core = lax.axis_index("c")
        sub = lax.axis_index("s")
        base = sub * PER_TILE

        def do(idx_hbm, out_hbm):
            def body(j, carry):
                off = base + j * CH
                pltpu.sync_copy(idx_hbm.at[pl.ds(off, CH)], idx_v)
                pltpu.async_copy(hid_hbm.at[idx_v], rows_v, sem).wait()
                pltpu.sync_copy(rows_v, out_hbm.at[pl.ds(off, CH)])
                return carry
            lax.fori_loop(0, NCHUNK, body, 0, unroll=False)

        @pl.when(core == 0)
        def _():
            do(idxp_hbm, outp_hbm)

        @pl.when(core == 1)
        def _():
            do(idxc_hbm, outc_hbm)

    return k(hid, idx_p, idx_c)


def _scatter_sc(sp_e, sc_e, idx_p, idx_c, zeros_nd):
    @functools.partial(
        pl.kernel,
        out_type=[
            jax.ShapeDtypeStruct((N_PAD, H2), _f32),
            jax.ShapeDtypeStruct((N_PAD, H2), _f32),
        ],
        mesh=_sc_mesh(),
        scratch_types=[
            pltpu.VMEM((CH,), jnp.int32),
            pltpu.VMEM((CH, H2), _f32),
            pltpu.VMEM_SHARED((N_PAD, H2), _f32),
        ],
    )
    def k(spe_hbm, sce_hbm, idxp_hbm, idxc_hbm, zeros_hbm,
          sump_hbm, sumc_hbm, idx_v, rows_v, acc):
        core = lax.axis_index("c")
        sub = lax.axis_index("s")
        r0 = sub * ROWS_PER_TILE

        def zbody(j, carry):
            rr = r0 + j * CH
            pltpu.sync_copy(zeros_hbm.at[pl.ds(rr, CH)], rows_v)
            pltpu.sync_copy(rows_v, acc.at[pl.ds(rr, CH)])
            return carry
        lax.fori_loop(0, ROWS_PER_TILE // CH, zbody, 0, unroll=False)
        plsc.subcore_barrier()

        def do(edges_hbm, idx_hbm):
            base = sub * PER_TILE

            def body(j, carry):
                off = base + j * CH
                pltpu.sync_copy(idx_hbm.at[pl.ds(off, CH)], idx_v)
                pltpu.sync_copy(edges_hbm.at[pl.ds(off, CH)], rows_v)
                pltpu.sync_copy(rows_v, acc.at[idx_v], add=True)
                return carry
            lax.fori_loop(0, NCHUNK, body, 0, unroll=False)

        @pl.when(core == 0)
        def _():
            do(spe_hbm, idxp_hbm)

        @pl.when(core == 1)
        def _():
            do(sce_hbm, idxc_hbm)

        plsc.subcore_barrier()

        def wbody(dst_hbm):
            def body(j, carry):
                rr = r0 + j * CH
                pltpu.sync_copy(acc.at[pl.ds(rr, CH)], rows_v)
                pltpu.sync_copy(rows_v, dst_hbm.at[pl.ds(rr, CH)])
                return carry
            lax.fori_loop(0, ROWS_PER_TILE // CH, body, 0, unroll=False)

        @pl.when(core == 0)
        def _():
            wbody(sump_hbm)

        @pl.when(core == 1)
        def _():
            wbody(sumc_hbm)

    return k(sp_e, sc_e, idx_p, idx_c, zeros_nd)


def kernel(batch_token, edge_p_node, edge_c_node, edge_p_indicate,
           edge_c_indicate, p_mask, c_mask, start_token, end_token,
           Wv1, bv1, Wv2, bv2, We1, be1, We2, be2,
           Wp1, bp1, Wp2, bp2, Wc1, bc1, Wc2, bc2,
           Wa1, ba1, Wa2, ba2):
    x = jnp.zeros((N_PAD, D), _f32).at[:N].set(batch_token)
    pm = jnp.zeros((N_PAD, 1), _f32).at[:N, 0].set(p_mask)
    cm = jnp.zeros((N_PAD, 1), _f32).at[:N, 0].set(c_mask)
    st = start_token.reshape(1, D)
    et = end_token.reshape(1, D)
    tp = edge_p_indicate.reshape(E, 1)
    tc = edge_c_indicate.reshape(E, 1)

    bv1_ = bv1.reshape(1, H1)
    bv2_ = bv2.reshape(1, H2)
    be1_ = be1.reshape(1, H1)
    be2_ = be2.reshape(1, H2)
    bp1_ = bp1.reshape(1, H1)
    bp2_ = bp2.reshape(1, H2)
    bc1_ = bc1.reshape(1, H1)
    bc2_ = bc2.reshape(1, H2)
    ba1_ = ba1.reshape(1, H1)
    ba2_ = ba2.reshape(1, H2)

    wp1c, wp1p, wp1e = Wp1[0:H2], Wp1[H2:2 * H2], Wp1[2 * H2:3 * H2]
    wc1p, wc1c, wc1e = Wc1[0:H2], Wc1[H2:2 * H2], Wc1[2 * H2:3 * H2]
    wa1h, wa1p, wa1c = Wa1[0:H2], Wa1[H2:2 * H2], Wa1[2 * H2:3 * H2]

    zeros_nd = jnp.zeros((N_PAD, H2), _f32)

    hidden = _node_mlp(x, Wv1, bv1_, Wv2, bv2_)
    prep, prec = _edge_pre(tp, tc, We1, be1_, We2, be2_,
                           wp1e, bp1_, wc1e, bc1_)
    ones_e = jnp.ones((E, H2), _f32)
    cntp, cntc = _scatter_sc(ones_e, ones_e, edge_p_node, edge_c_node,
                             zeros_nd)

    for _ in range(NUM_HOPS):
        hp, hc = _gather_sc(hidden, edge_p_node, edge_c_node)
        sp_e, sc_e = _edge_mlp(hp, hc, prep, prec,
                               wp1c, wp1p, Wp2, bp2_,
                               wc1p, wc1c, Wc2, bc2_)
        sump, sumc = _scatter_sc(sp_e, sc_e, edge_p_node, edge_c_node,
                                 zeros_nd)
        hidden = _aggr(hidden, sump, sumc, cntp, cntc, pm, cm, st, et,
                       wa1h, wa1p, wa1c, ba1_, Wa2, ba2_)

    return hidden[:N]


# speedup vs baseline: 2.5190x; 1.3370x over previous
import functools

import jax
import jax.numpy as jnp
from jax import lax
from jax.experimental import pallas as pl
from jax.experimental.pallas import tpu as pltpu
from jax.experimental.pallas import tpu_sc as plsc

N = 10000
E = 320000
D = 128
H1 = 256
H2 = 128
NUM_HOPS = 3

N_PAD = 10240
NS = 16
PER_TILE = E // NS
CH = 80
NCHUNK = PER_TILE // CH
ROWS_PER_TILE = N_PAD // NS

BLK_N = 2048
BLK_E = 1280

_f32 = jnp.float32


def _node_mlp_body(x_ref, w1_ref, b1_ref, w2_ref, b2_ref, o_ref):
    h = jnp.maximum(
        jnp.dot(x_ref[...], w1_ref[...], preferred_element_type=_f32)
        + b1_ref[...], 0.0)
    o_ref[...] = jnp.maximum(
        jnp.dot(h, w2_ref[...], preferred_element_type=_f32)
        + b2_ref[...], 0.0)


def _node_mlp(x, w1, b1, w2, b2):
    n = x.shape[0]
    grid = n // BLK_N
    return pl.pallas_call(
        _node_mlp_body,
        grid=(grid,),
        in_specs=[
            pl.BlockSpec((BLK_N, D), lambda i: (i, 0)),
            pl.BlockSpec((D, H1), lambda i: (0, 0)),
            pl.BlockSpec((1, H1), lambda i: (0, 0)),
            pl.BlockSpec((H1, H2), lambda i: (0, 0)),
            pl.BlockSpec((1, H2), lambda i: (0, 0)),
        ],
        out_specs=pl.BlockSpec((BLK_N, H2), lambda i: (i, 0)),
        out_shape=jax.ShapeDtypeStruct((n, H2), _f32),
    )(x, w1, b1, w2, b2)


def _edge_pre_body(tp_ref, tc_ref, we1_ref, be1_ref, we2_ref, be2_ref,
                   wpe_ref, bp1_ref, wce_ref, bc1_ref, prep_ref, prec_ref):
    def one(t_ref, we_ref, b_ref):
        h = jnp.maximum(t_ref[...] * we1_ref[...] + be1_ref[...], 0.0)
        e = jnp.maximum(
            jnp.dot(h, we2_ref[...], preferred_element_type=_f32)
            + be2_ref[...], 0.0)
        return jnp.dot(e, we_ref[...], preferred_element_type=_f32) + b_ref[...]
    prep_ref[...] = one(tp_ref, wpe_ref, bp1_ref)
    prec_ref[...] = one(tc_ref, wce_ref, bc1_ref)


def _edge_pre(tp, tc, we1, be1, we2, be2, wpe, bp1, wce, bc1):
    grid = E // BLK_E
    return pl.pallas_call(
        _edge_pre_body,
        grid=(grid,),
        in_specs=[
            pl.BlockSpec((BLK_E, 1), lambda i: (i, 0)),
            pl.BlockSpec((BLK_E, 1), lambda i: (i, 0)),
            pl.BlockSpec((1, H1), lambda i: (0, 0)),
            pl.BlockSpec((1, H1), lambda i: (0, 0)),
            pl.BlockSpec((H1, H2), lambda i: (0, 0)),
            pl.BlockSpec((1, H2), lambda i: (0, 0)),
            pl.BlockSpec((H2, H1), lambda i: (0, 0)),
            pl.BlockSpec((1, H1), lambda i: (0, 0)),
            pl.BlockSpec((H2, H1), lambda i: (0, 0)),
            pl.BlockSpec((1, H1), lambda i: (0, 0)),
        ],
        out_specs=[
            pl.BlockSpec((BLK_E, H1), lambda i: (i, 0)),
            pl.BlockSpec((BLK_E, H1), lambda i: (i, 0)),
        ],
        out_shape=[
            jax.ShapeDtypeStruct((E, H1), _f32),
            jax.ShapeDtypeStruct((E, H1), _f32),
        ],
    )(tp, tc, we1, be1, we2, be2, wpe, bp1, wce, bc1)


def _edge_mlp_body(hp_ref, hc_ref, prep_ref, prec_ref,
                   wp1c_ref, wp1p_ref, wp2_ref, bp2_ref,
                   wc1p_ref, wc1c_ref, wc2_ref, bc2_ref,
                   sp_ref, sc_ref):
    hp = hp_ref[...]
    hc = hc_ref[...]
    h1p = jnp.maximum(
        jnp.dot(hc, wp1c_ref[...], preferred_element_type=_f32)
        + jnp.dot(hp, wp1p_ref[...], preferred_element_type=_f32)
        + prep_ref[...], 0.0)
    sp_ref[...] = jnp.maximum(
        jnp.dot(h1p, wp2_ref[...], preferred_element_type=_f32)
        + bp2_ref[...], 0.0)
    h1c = jnp.maximum(
        jnp.dot(hp, wc1p_ref[...], preferred_element_type=_f32)
        + jnp.dot(hc, wc1c_ref[...], preferred_element_type=_f32)
        + prec_ref[...], 0.0)
    sc_ref[...] = jnp.maximum(
        jnp.dot(h1c, wc2_ref[...], preferred_element_type=_f32)
        + bc2_ref[...], 0.0)


def _edge_mlp(hp, hc, prep, prec, wp1c, wp1p, wp2, bp2, wc1p, wc1c, wc2, bc2):
    grid = E // BLK_E
    return pl.pallas_call(
        _edge_mlp_body,
        grid=(grid,),
        in_specs=[
            pl.BlockSpec((BLK_E, H2), lambda i: (i, 0)),
            pl.BlockSpec((BLK_E, H2), lambda i: (i, 0)),
            pl.BlockSpec((BLK_E, H1), lambda i: (i, 0)),
            pl.BlockSpec((BLK_E, H1), lambda i: (i, 0)),
            pl.BlockSpec((H2, H1), lambda i: (0, 0)),
            pl.BlockSpec((H2, H1), lambda i: (0, 0)),
            pl.BlockSpec((H1, H2), lambda i: (0, 0)),
            pl.BlockSpec((1, H2), lambda i: (0, 0)),
            pl.BlockSpec((H2, H1), lambda i: (0, 0)),
            pl.BlockSpec((H2, H1), lambda i: (0, 0)),
            pl.BlockSpec((H1, H2), lambda i: (0, 0)),
            pl.BlockSpec((1, H2), lambda i: (0, 0)),
        ],
        out_specs=[
            pl.BlockSpec((BLK_E, H2), lambda i: (i, 0)),
            pl.BlockSpec((BLK_E, H2), lambda i: (i, 0)),
        ],
        out_shape=[
            jax.ShapeDtypeStruct((E, H2), _f32),
            jax.ShapeDtypeStruct((E, H2), _f32),
        ],
    )(hp, hc, prep, prec, wp1c, wp1p, wp2, bp2, wc1p, wc1c, wc2, bc2)


def _aggr_body(hid_ref, sump_ref, sumc_ref, cntp_ref, cntc_ref,
               pm_ref, cm_ref, st_ref, et_ref,
               wa1h_ref, wa1p_ref, wa1c_ref, ba1_ref, wa2_ref, ba2_ref,
               o_ref):
    hid = hid_ref[...]
    s_p = sump_ref[...] / jnp.maximum(cntp_ref[:, :1], 1.0) \
        + pm_ref[...] * st_ref[...]
    s_c = sumc_ref[...] / jnp.maximum(cntc_ref[:, :1], 1.0) \
        + cm_ref[...] * et_ref[...]
    h1 = jnp.maximum(
        jnp.dot(hid, wa1h_ref[...], preferred_element_type=_f32)
        + jnp.dot(s_p, wa1p_ref[...], preferred_element_type=_f32)
        + jnp.dot(s_c, wa1c_ref[...], preferred_element_type=_f32)
        + ba1_ref[...], 0.0)
    o_ref[...] = hid + jnp.maximum(
        jnp.dot(h1, wa2_ref[...], preferred_element_type=_f32)
        + ba2_ref[...], 0.0)


def _aggr(hid, sump, sumc, cntp, cntc, pm, cm, st, et,
          wa1h, wa1p, wa1c, ba1, wa2, ba2):
    grid = N_PAD // BLK_N
    return pl.pallas_call(
        _aggr_body,
        grid=(grid,),
        in_specs=[
            pl.BlockSpec((BLK_N, H2), lambda i: (i, 0)),
            pl.BlockSpec((BLK_N, H2), lambda i: (i, 0)),
            pl.BlockSpec((BLK_N, H2), lambda i: (i, 0)),
            pl.BlockSpec((BLK_N, H2), lambda i: (i, 0)),
            pl.BlockSpec((BLK_N, H2), lambda i: (i, 0)),
            pl.BlockSpec((BLK_N, 1), lambda i: (i, 0)),
            pl.BlockSpec((BLK_N, 1), lambda i: (i, 0)),
            pl.BlockSpec((1, D), lambda i: (0, 0)),
            pl.BlockSpec((1, D), lambda i: (0, 0)),
            pl.BlockSpec((H2, H1), lambda i: (0, 0)),
            pl.BlockSpec((H2, H1), lambda i: (0, 0)),
            pl.BlockSpec((H2, H1), lambda i: (0, 0)),
            pl.BlockSpec((1, H1), lambda i: (0, 0)),
            pl.BlockSpec((H1, H2), lambda i: (0, 0)),
            pl.BlockSpec((1, H2), lambda i: (0, 0)),
        ],
        out_specs=pl.BlockSpec((BLK_N, H2), lambda i: (i, 0)),
        out_shape=jax.ShapeDtypeStruct((N_PAD, H2), _f32),
    )(hid, sump, sumc, cntp, cntc, pm, cm, st, et,
      wa1h, wa1p, wa1c, ba1, wa2, ba2)


def _sc_mesh():
    return plsc.VectorSubcoreMesh(core_axis_name="c", subcore_axis_name="s",
                                  num_cores=2, num_subcores=NS)


K_G = 5
NG = NCHUNK // K_G
NGH = NG // 2


def _gather_sc(hid, idx_p, idx_c):
    @functools.partial(
        pl.kernel,
        out_type=[
            jax.ShapeDtypeStruct((E, H2), _f32),
            jax.ShapeDtypeStruct((E, H2), _f32),
        ],
        mesh=_sc_mesh(),
        scratch_types=[
            pltpu.VMEM((K_G, CH), jnp.int32),
            pltpu.VMEM((K_G, CH), jnp.int32),
            pltpu.VMEM((K_G, CH, H2), _f32),
            pltpu.VMEM((K_G, CH, H2), _f32),
            pltpu.SemaphoreType.DMA,
            pltpu.SemaphoreType.DMA,
            pltpu.SemaphoreType.DMA,
            pltpu.SemaphoreType.DMA,
        ],
    )
    def k(hid_hbm, idxp_hbm, idxc_hbm, outp_hbm, outc_hbm,
          idxa, idxb, rowsa, rowsb, gsema, gsemb, wsema, wsemb):
        core = lax.axis_index("c")
        sub = lax.axis_index("s")
        base = sub * PER_TILE

        def do(idx_hbm, out_hbm):
            def fire_gathers(idxv, rowsv, gsem, g):
                for b in range(K_G):
                    off = base + (g * K_G + b) * CH
                    pltpu.sync_copy(idx_hbm.at[pl.ds(off, CH)], idxv.at[b])
                    pltpu.async_copy(hid_hbm.at[idxv.at[b]], rowsv.at[b],
                                     gsem)

            def drain_gathers(idxv, rowsv, gsem):
                for b in range(K_G):
                    pltpu.make_async_copy(hid_hbm.at[idxv.at[b]],
                                          rowsv.at[b], gsem).wait()

            def fire_writes(rowsv, wsem, g):
                for b in range(K_G):
                    off = base + (g * K_G + b) * CH
                    pltpu.async_copy(rowsv.at[b],
                                     out_hbm.at[pl.ds(off, CH)], wsem)

            def drain_writes(rowsv, wsem, g):
                for b in range(K_G):
                    off = base + (g * K_G + b) * CH
                    pltpu.make_async_copy(rowsv.at[b],
                                          out_hbm.at[pl.ds(off, CH)],
                                          wsem).wait()

            fire_gathers(idxa, rowsa, gsema, 0)

            def body(i, carry):
                g0 = 2 * i

                @pl.when(i > 0)
                def _():
                    drain_writes(rowsb, wsemb, g0 - 1)
                fire_gathers(idxb, rowsb, gsemb, g0 + 1)
                drain_gathers(idxa, rowsa, gsema)
                fire_writes(rowsa, wsema, g0)
                drain_writes(rowsa, wsema, g0)

                @pl.when(i < NGH - 1)
                def _():
                    fire_gathers(idxa, rowsa, gsema, g0 + 2)
                drain_gathers(idxb, rowsb, gsemb)
                fire_writes(rowsb, wsemb, g0 + 1)
                return carry
            lax.fori_loop(0, NGH, body, 0, unroll=False)
            drain_writes(rowsb, wsemb, NG - 1)

        @pl.when(core == 0)
        def _():
            do(idxp_hbm, outp_hbm)

        @pl.when(core == 1)
        def _():
            do(idxc_hbm, outc_hbm)

    return k(hid, idx_p, idx_c)


def _scatter_sc(sp_e, sc_e, idx_p, idx_c, zeros_nd):
    @functools.partial(
        pl.kernel,
        out_type=[
            jax.ShapeDtypeStruct((N_PAD, H2), _f32),
            jax.ShapeDtypeStruct((N_PAD, H2), _f32),
        ],
        mesh=_sc_mesh(),
        scratch_types=[
            pltpu.VMEM((CH,), jnp.int32),
            pltpu.VMEM((CH,), jnp.int32),
            pltpu.VMEM((CH, H2), _f32),
            pltpu.VMEM((CH, H2), _f32),
            pltpu.SemaphoreType.DMA,
            pltpu.SemaphoreType.DMA,
            pltpu.VMEM_SHARED((N_PAD, H2), _f32),
        ],
    )
    def k(spe_hbm, sce_hbm, idxp_hbm, idxc_hbm, zeros_hbm,
          sump_hbm, sumc_hbm, idx0, idx1, rows0, rows1, rsem0, rsem1, acc):
        core = lax.axis_index("c")
        sub = lax.axis_index("s")
        r0 = sub * ROWS_PER_TILE

        def zbody(j, carry):
            rr = r0 + j * CH
            pltpu.sync_copy(zeros_hbm.at[pl.ds(rr, CH)], rows0)
            pltpu.sync_copy(rows0, acc.at[pl.ds(rr, CH)])
            return carry
        lax.fori_loop(0, ROWS_PER_TILE // CH, zbody, 0, unroll=False)
        plsc.subcore_barrier()

        def do(edges_hbm, idx_hbm):
            base = sub * PER_TILE
            pltpu.sync_copy(idx_hbm.at[pl.ds(base, CH)], idx0)
            pltpu.async_copy(edges_hbm.at[pl.ds(base, CH)], rows0, rsem0)

            def body(i, carry):
                j = 2 * i
                off1 = base + (j + 1) * CH
                pltpu.sync_copy(idx_hbm.at[pl.ds(off1, CH)], idx1)
                pltpu.async_copy(edges_hbm.at[pl.ds(off1, CH)], rows1, rsem1)
                pltpu.make_async_copy(edges_hbm.at[pl.ds(off1, CH)], rows0,
                                      rsem0).wait()
                pltpu.sync_copy(rows0, acc.at[idx0], add=True)

                @pl.when(i < NCHUNK // 2 - 1)
                def _():
                    off2 = base + (j + 2) * CH
                    pltpu.sync_copy(idx_hbm.at[pl.ds(off2, CH)], idx0)
                    pltpu.async_copy(edges_hbm.at[pl.ds(off2, CH)], rows0,
                                     rsem0)
                pltpu.make_async_copy(edges_hbm.at[pl.ds(off1, CH)], rows1,
                                      rsem1).wait()
                pltpu.sync_copy(rows1, acc.at[idx1], add=True)
                return carry
            lax.fori_loop(0, NCHUNK // 2, body, 0, unroll=False)

        @pl.when(core == 0)
        def _():
            do(spe_hbm, idxp_hbm)

        @pl.when(core == 1)
        def _():
            do(sce_hbm, idxc_hbm)

        plsc.subcore_barrier()

        def wbody(dst_hbm):
            def body(j, carry):
                rr = r0 + j * CH
                pltpu.sync_copy(acc.at[pl.ds(rr, CH)], rows0)
                pltpu.sync_copy(rows0, dst_hbm.at[pl.ds(rr, CH)])
                return carry
            lax.fori_loop(0, ROWS_PER_TILE // CH, body, 0, unroll=False)

        @pl.when(core == 0)
        def _():
            wbody(sump_hbm)

        @pl.when(core == 1)
        def _():
            wbody(sumc_hbm)

    return k(sp_e, sc_e, idx_p, idx_c, zeros_nd)


def _counts_sc(idx_p, idx_c, ones_rows, zeros_nd):
    @functools.partial(
        pl.kernel,
        out_type=[
            jax.ShapeDtypeStruct((N_PAD, H2), _f32),
            jax.ShapeDtypeStruct((N_PAD, H2), _f32),
        ],
        mesh=_sc_mesh(),
        scratch_types=[
            pltpu.VMEM((CH,), jnp.int32),
            pltpu.VMEM((CH, H2), _f32),
            pltpu.VMEM_SHARED((N_PAD, H2), _f32),
        ],
    )
    def k(idxp_hbm, idxc_hbm, ones_hbm, zeros_hbm,
          cntp_hbm, cntc_hbm, idx_v, rows_v, acc):
        core = lax.axis_index("c")
        sub = lax.axis_index("s")
        r0 = sub * ROWS_PER_TILE

        def zbody(j, carry):
            rr = r0 + j * CH
            pltpu.sync_copy(zeros_hbm.at[pl.ds(rr, CH)], rows_v)
            pltpu.sync_copy(rows_v, acc.at[pl.ds(rr, CH)])
            return carry
        lax.fori_loop(0, ROWS_PER_TILE // CH, zbody, 0, unroll=False)
        plsc.subcore_barrier()

        pltpu.sync_copy(ones_hbm, rows_v)

        def do(idx_hbm):
            base = sub * PER_TILE

            def body(j, carry):
                off = base + j * CH
                pltpu.sync_copy(idx_hbm.at[pl.ds(off, CH)], idx_v)
                pltpu.sync_copy(rows_v, acc.at[idx_v], add=True)
                return carry
            lax.fori_loop(0, NCHUNK, body, 0, unroll=False)

        @pl.when(core == 0)
        def _():
            do(idxp_hbm)

        @pl.when(core == 1)
        def _():
            do(idxc_hbm)

        plsc.subcore_barrier()

        def wbody(dst_hbm):
            def body(j, carry):
                rr = r0 + j * CH
                pltpu.sync_copy(acc.at[pl.ds(rr, CH)], rows_v)
                pltpu.sync_copy(rows_v, dst_hbm.at[pl.ds(rr, CH)])
                return carry
            lax.fori_loop(0, ROWS_PER_TILE // CH, body, 0, unroll=False)

        @pl.when(core == 0)
        def _():
            wbody(cntp_hbm)

        @pl.when(core == 1)
        def _():
            wbody(cntc_hbm)

    return k(idx_p, idx_c, ones_rows, zeros_nd)


def kernel(batch_token, edge_p_node, edge_c_node, edge_p_indicate,
           edge_c_indicate, p_mask, c_mask, start_token, end_token,
           Wv1, bv1, Wv2, bv2, We1, be1, We2, be2,
           Wp1, bp1, Wp2, bp2, Wc1, bc1, Wc2, bc2,
           Wa1, ba1, Wa2, ba2):
    x = jnp.zeros((N_PAD, D), _f32).at[:N].set(batch_token)
    pm = jnp.zeros((N_PAD, 1), _f32).at[:N, 0].set(p_mask)
    cm = jnp.zeros((N_PAD, 1), _f32).at[:N, 0].set(c_mask)
    st = start_token.reshape(1, D)
    et = end_token.reshape(1, D)
    tp = edge_p_indicate.reshape(E, 1)
    tc = edge_c_indicate.reshape(E, 1)

    bv1_ = bv1.reshape(1, H1)
    bv2_ = bv2.reshape(1, H2)
    be1_ = be1.reshape(1, H1)
    be2_ = be2.reshape(1, H2)
    bp1_ = bp1.reshape(1, H1)
    bp2_ = bp2.reshape(1, H2)
    bc1_ = bc1.reshape(1, H1)
    bc2_ = bc2.reshape(1, H2)
    ba1_ = ba1.reshape(1, H1)
    ba2_ = ba2.reshape(1, H2)

    wp1c, wp1p, wp1e = Wp1[0:H2], Wp1[H2:2 * H2], Wp1[2 * H2:3 * H2]
    wc1p, wc1c, wc1e = Wc1[0:H2], Wc1[H2:2 * H2], Wc1[2 * H2:3 * H2]
    wa1h, wa1p, wa1c = Wa1[0:H2], Wa1[H2:2 * H2], Wa1[2 * H2:3 * H2]

    zeros_nd = jnp.zeros((N_PAD, H2), _f32)

    hidden = _node_mlp(x, Wv1, bv1_, Wv2, bv2_)
    prep, prec = _edge_pre(tp, tc, We1, be1_, We2, be2_,
                           wp1e, bp1_, wc1e, bc1_)
    ones_rows = jnp.ones((CH, H2), _f32)
    cntp, cntc = _counts_sc(edge_p_node, edge_c_node, ones_rows, zeros_nd)

    for _ in range(NUM_HOPS):
        hp, hc = _gather_sc(hidden, edge_p_node, edge_c_node)
        sp_e, sc_e = _edge_mlp(hp, hc, prep, prec,
                               wp1c, wp1p, Wp2, bp2_,
                               wc1p, wc1c, Wc2, bc2_)
        sump, sumc = _scatter_sc(sp_e, sc_e, edge_p_node, edge_c_node,
                                 zeros_nd)
        hidden = _aggr(hidden, sump, sumc, cntp, cntc, pm, cm, st, et,
                       wa1h, wa1p, wa1c, ba1_, Wa2, ba2_)

    return hidden[:N]
